# speedup vs baseline: 1.6956x; 1.6956x over previous
import jax
import jax.numpy as jnp
from jax.experimental import pallas as pl

BT = 512


def _add_kernel(x_ref, pos_ref, o_ref):
    o_ref[...] = x_ref[...] + pos_ref[...]


def kernel(x, pos_table):
    B, T, D = x.shape
    num_t = T // BT
    grid = (num_t, B)
    return pl.pallas_call(
        _add_kernel,
        grid=grid,
        in_specs=[
            pl.BlockSpec((1, BT, D), lambda t, b: (b, t, 0)),
            pl.BlockSpec((BT, D), lambda t, b: (t, 0)),
        ],
        out_specs=pl.BlockSpec((1, BT, D), lambda t, b: (b, t, 0)),
        out_shape=jax.ShapeDtypeStruct((B, T, D), x.dtype),
    )(x, pos_table)


# speedup vs baseline: 1.8752x; 1.1059x over previous
import jax
import jax.numpy as jnp
from jax.experimental import pallas as pl
from jax.experimental.pallas import tpu as pltpu

BT = 1024


def _add_kernel(x_ref, pos_ref, o_ref):
    o_ref[...] = x_ref[...] + pos_ref[...]


def kernel(x, pos_table):
    B, T, D = x.shape
    num_t = T // BT
    grid = (num_t, B)
    return pl.pallas_call(
        _add_kernel,
        grid=grid,
        in_specs=[
            pl.BlockSpec((1, BT, D), lambda t, b: (b, t, 0)),
            pl.BlockSpec((BT, D), lambda t, b: (t, 0)),
        ],
        out_specs=pl.BlockSpec((1, BT, D), lambda t, b: (b, t, 0)),
        out_shape=jax.ShapeDtypeStruct((B, T, D), x.dtype),
        compiler_params=pltpu.CompilerParams(
            dimension_semantics=("parallel", "arbitrary"),
        ),
    )(x, pos_table)


# speedup vs baseline: 1.9950x; 1.0639x over previous
import jax
import jax.numpy as jnp
from jax.experimental import pallas as pl
from jax.experimental.pallas import tpu as pltpu

BT = 2048


def _add_kernel(x_ref, pos_ref, o_ref):
    o_ref[...] = x_ref[...] + pos_ref[...]


def kernel(x, pos_table):
    B, T, D = x.shape
    num_t = T // BT
    grid = (num_t, B)
    return pl.pallas_call(
        _add_kernel,
        grid=grid,
        in_specs=[
            pl.BlockSpec((1, BT, D), lambda t, b: (b, t, 0)),
            pl.BlockSpec((BT, D), lambda t, b: (t, 0)),
        ],
        out_specs=pl.BlockSpec((1, BT, D), lambda t, b: (b, t, 0)),
        out_shape=jax.ShapeDtypeStruct((B, T, D), x.dtype),
        compiler_params=pltpu.CompilerParams(
            dimension_semantics=("parallel", "arbitrary"),
        ),
    )(x, pos_table)


# speedup vs baseline: 1.9959x; 1.0004x over previous
import jax
import jax.numpy as jnp
from jax.experimental import pallas as pl
from jax.experimental.pallas import tpu as pltpu

BT = 2048


def _add_kernel(x_ref, pos_ref, o_ref):
    o_ref[...] = x_ref[...] + pos_ref[...]


def kernel(x, pos_table):
    B, T, D = x.shape
    num_t = T // BT
    grid = (num_t, B)
    return pl.pallas_call(
        _add_kernel,
        grid=grid,
        in_specs=[
            pl.BlockSpec((1, BT, D), lambda t, b: (b, t, 0)),
            pl.BlockSpec((BT, D), lambda t, b: (t, 0)),
        ],
        out_specs=pl.BlockSpec((1, BT, D), lambda t, b: (b, t, 0)),
        out_shape=jax.ShapeDtypeStruct((B, T, D), x.dtype),
        compiler_params=pltpu.CompilerParams(
            dimension_semantics=("parallel", "parallel"),
            vmem_limit_bytes=128 * 1024 * 1024,
        ),
    )(x, pos_table)
